# baseline (speedup 1.0000x reference)
import functools

import jax
import jax.numpy as jnp
from jax import lax
from jax.experimental import pallas as pl
from jax.experimental.pallas import tpu as pltpu
from jax.experimental.pallas import tpu_sc as plsc

LANES = 16


@functools.partial(jax.jit, static_argnames=("B", "T", "D"))
def _embed(x, token_table, posT_tiled, B, T, D):
    NC, NS = 2, 16
    NW = NC * NS
    W = 128
    B_PER_W = B // NW
    NTC = T // W
    DT = D // 8

    mesh = plsc.VectorSubcoreMesh(core_axis_name="c", subcore_axis_name="s")

    @functools.partial(
        pl.kernel,
        mesh=mesh,
        compiler_params=pltpu.CompilerParams(
            use_tc_tiling_on_sc=False, needs_layout_passes=False
        ),
        out_type=jax.ShapeDtypeStruct((B, DT, NTC, 8, W), jnp.float32),
        scratch_types=[
            pltpu.VMEM((B_PER_W, W), jnp.int32),
            pltpu.VMEM((W, D), jnp.float32),
            pltpu.VMEM((DT, 8, W), jnp.float32),
            pltpu.VMEM((DT, 8, W), jnp.float32),
            pltpu.SemaphoreType.DMA,
        ],
    )
    def k(x_hbm, tok_hbm, posT_hbm, out_hbm, idx_v, rows_v, pos_v, out_v, sem):
        wid = lax.axis_index("s") * NC + lax.axis_index("c")
        b0 = wid * B_PER_W
        iota = lax.iota(jnp.int32, LANES)

        @pl.loop(0, NTC)
        def _(tc):
            pltpu.sync_copy(posT_hbm.at[:, tc], pos_v)
            pltpu.sync_copy(
                x_hbm.at[pl.ds(b0, B_PER_W), pl.ds(tc * W, W)], idx_v
            )

            @pl.loop(0, B_PER_W)
            def _(i):
                pltpu.async_copy(tok_hbm.at[idx_v.at[i]], rows_v, sem).wait()

                @pl.loop(0, DT)
                def _(dt):
                    for dr in range(8):
                        dcol = jnp.full((LANES,), 0, jnp.int32) + (
                            dt * 8 + dr
                        )
                        for tg in range(0, W, LANES):
                            rvec = tg + iota
                            v = plsc.load_gather(rows_v, [rvec, dcol])
                            out_v[dt, dr, pl.ds(tg, LANES)] = (
                                v + pos_v[dt, dr, pl.ds(tg, LANES)]
                            )

                pltpu.sync_copy(out_v, out_hbm.at[b0 + i, :, tc])

    return k(x, token_table, posT_tiled)


def kernel(x, token_table, pos_table):
    B, T = x.shape
    D = token_table.shape[1]
    posT = (
        pos_table.T.reshape(D // 8, 8, T // 128, 128).transpose(0, 2, 1, 3)
    )
    out5 = _embed(x, token_table, posT, B, T, D)
    out = (
        out5.transpose(0, 1, 3, 2, 4)
        .reshape(B, D, T)
        .transpose(0, 2, 1)
    )
    return out


# speedup vs baseline: 1.1488x; 1.1488x over previous
import functools

import jax
import jax.numpy as jnp
from jax import lax
from jax.experimental import pallas as pl
from jax.experimental.pallas import tpu as pltpu
from jax.experimental.pallas import tpu_sc as plsc

LANES = 16


@functools.partial(jax.jit, static_argnames=("B", "T", "D"))
def _embed(x, token_table, pos_table, B, T, D):
    NC, NS = 2, 16
    NW = NC * NS
    W = 128
    B_PER_W = B // NW
    NTC = T // W
    DT = D // 8

    mesh = plsc.VectorSubcoreMesh(core_axis_name="c", subcore_axis_name="s")

    @functools.partial(
        pl.kernel,
        mesh=mesh,
        compiler_params=pltpu.CompilerParams(
            use_tc_tiling_on_sc=False, needs_layout_passes=False
        ),
        out_type=jax.ShapeDtypeStruct((B, DT, NTC, 8, W), jnp.float32),
        scratch_types=[
            pltpu.VMEM((B_PER_W, W), jnp.int32),
            pltpu.VMEM((W, D), jnp.float32),
            pltpu.VMEM((W, D), jnp.float32),
            pltpu.VMEM((DT, 8, W), jnp.float32),
            pltpu.SemaphoreType.DMA,
        ],
    )
    def k(x_hbm, tok_hbm, pos_hbm, out_hbm, idx_v, rows_v, pos_v, out_v, sem):
        wid = lax.axis_index("s") * NC + lax.axis_index("c")
        b0 = wid * B_PER_W
        iota = lax.iota(jnp.int32, LANES)
        dt_vecs = [(c + iota) >> 3 for c in range(0, D, LANES)]
        dr_vecs = [(c + iota) & 7 for c in range(0, D, LANES)]

        @pl.loop(0, NTC)
        def _(tc):
            pltpu.sync_copy(pos_hbm.at[pl.ds(tc * W, W)], pos_v)
            pltpu.sync_copy(
                x_hbm.at[pl.ds(b0, B_PER_W), pl.ds(tc * W, W)], idx_v
            )

            @pl.loop(0, B_PER_W)
            def _(i):
                pltpu.async_copy(tok_hbm.at[idx_v.at[i]], rows_v, sem).wait()

                @pl.loop(0, W)
                def _(t):
                    tvec = jnp.full((LANES,), 0, jnp.int32) + t
                    for g, c in enumerate(range(0, D, LANES)):
                        v = (
                            rows_v[t, pl.ds(c, LANES)]
                            + pos_v[t, pl.ds(c, LANES)]
                        )
                        plsc.store_scatter(
                            out_v, [dt_vecs[g], dr_vecs[g], tvec], v
                        )

                pltpu.sync_copy(out_v, out_hbm.at[b0 + i, :, tc])

    return k(x, token_table, pos_table)


def kernel(x, token_table, pos_table):
    B, T = x.shape
    D = token_table.shape[1]
    out5 = _embed(x, token_table, pos_table, B, T, D)
    out = (
        out5.transpose(0, 1, 3, 2, 4)
        .reshape(B, D, T)
        .transpose(0, 2, 1)
    )
    return out


# speedup vs baseline: 1.3957x; 1.2149x over previous
import functools

import jax
import jax.numpy as jnp
from jax import lax
from jax.experimental import pallas as pl
from jax.experimental.pallas import tpu as pltpu
from jax.experimental.pallas import tpu_sc as plsc

LANES = 16


@functools.partial(jax.jit, static_argnames=("B", "T", "D"))
def _embed(x, token_table, pos_table, B, T, D):
    NC, NS = 2, 16
    NW = NC * NS
    W = 128
    B_PER_W = B // NW
    NTC = T // W
    DT = D // 8
    TUNROLL = 4

    mesh = plsc.VectorSubcoreMesh(core_axis_name="c", subcore_axis_name="s")

    @functools.partial(
        pl.kernel,
        mesh=mesh,
        compiler_params=pltpu.CompilerParams(
            use_tc_tiling_on_sc=False, needs_layout_passes=False
        ),
        out_type=jax.ShapeDtypeStruct((B, DT, NTC, 8, W), jnp.float32),
        scratch_types=[
            pltpu.VMEM((B_PER_W, W), jnp.int32),
            pltpu.VMEM((W, D), jnp.float32),
            pltpu.VMEM((W, D), jnp.float32),
            pltpu.VMEM((W, D), jnp.float32),
            pltpu.VMEM((DT, 8, W), jnp.float32),
            pltpu.VMEM((DT, 8, W), jnp.float32),
            pltpu.SemaphoreType.DMA,
            pltpu.SemaphoreType.DMA,
            pltpu.SemaphoreType.DMA,
            pltpu.SemaphoreType.DMA,
        ],
    )
    def k(
        x_hbm, tok_hbm, pos_hbm, out_hbm,
        idx_v, rows0, rows1, pos_v, out0, out1,
        sg0, sg1, st0, st1,
    ):
        wid = lax.axis_index("s") * NC + lax.axis_index("c")
        b0 = wid * B_PER_W
        iota = lax.iota(jnp.int32, LANES)
        dt_vecs = [(c + iota) >> 3 for c in range(0, D, LANES)]
        dr_vecs = [(c + iota) & 7 for c in range(0, D, LANES)]
        rows = (rows0, rows1)
        outs = (out0, out1)
        sgs = (sg0, sg1)
        sts = (st0, st1)

        def compute(p):
            rv, ov = rows[p], outs[p]

            @pl.loop(0, W, step=TUNROLL)
            def _(t):
                for u in range(TUNROLL):
                    tvec = jnp.full((LANES,), 0, jnp.int32) + (t + u)
                    for g, c in enumerate(range(0, D, LANES)):
                        v = (
                            rv[t + u, pl.ds(c, LANES)]
                            + pos_v[t + u, pl.ds(c, LANES)]
                        )
                        plsc.store_scatter(
                            ov, [dt_vecs[g], dr_vecs[g], tvec], v
                        )

        @pl.loop(0, NTC)
        def _(tc):
            pltpu.sync_copy(pos_hbm.at[pl.ds(tc * W, W)], pos_v)
            pltpu.sync_copy(
                x_hbm.at[pl.ds(b0, B_PER_W), pl.ds(tc * W, W)], idx_v
            )
            pltpu.make_async_copy(
                tok_hbm.at[idx_v.at[0]], rows[0], sgs[0]
            ).start()

            @pl.loop(0, B_PER_W // 2)
            def _(i2):
                for p in range(2):
                    i = 2 * i2 + p
                    q = 1 - p
                    pltpu.make_async_copy(
                        tok_hbm.at[idx_v.at[i]], rows[p], sgs[p]
                    ).wait()

                    @pl.when(i < B_PER_W - 1)
                    def _():
                        pltpu.make_async_copy(
                            tok_hbm.at[idx_v.at[i + 1]], rows[q], sgs[q]
                        ).start()

                    @pl.when(i >= 2)
                    def _():
                        pltpu.make_async_copy(
                            outs[p], out_hbm.at[b0 + i - 2, :, tc], sts[p]
                        ).wait()

                    compute(p)
                    pltpu.make_async_copy(
                        outs[p], out_hbm.at[b0 + i, :, tc], sts[p]
                    ).start()

            pltpu.make_async_copy(
                outs[0], out_hbm.at[b0 + B_PER_W - 2, :, tc], sts[0]
            ).wait()
            pltpu.make_async_copy(
                outs[1], out_hbm.at[b0 + B_PER_W - 1, :, tc], sts[1]
            ).wait()

    return k(x, token_table, pos_table)


def kernel(x, token_table, pos_table):
    B, T = x.shape
    D = token_table.shape[1]
    out5 = _embed(x, token_table, pos_table, B, T, D)
    out = (
        out5.transpose(0, 1, 3, 2, 4)
        .reshape(B, D, T)
        .transpose(0, 2, 1)
    )
    return out


# speedup vs baseline: 2.0876x; 1.4958x over previous
import functools

import jax
import jax.numpy as jnp
from jax import lax
from jax.experimental import pallas as pl
from jax.experimental.pallas import tpu as pltpu
from jax.experimental.pallas import tpu_sc as plsc

LANES = 16


@functools.partial(jax.jit, static_argnames=("B", "T", "D"))
def _embed(x, token_table, pos_table, B, T, D):
    NC, NS = 2, 16
    NW = NC * NS
    W = 128
    B_PER_W = B // NW
    NTC = T // W
    DT = D // 8
    TUNROLL = 4

    mesh = plsc.VectorSubcoreMesh(core_axis_name="c", subcore_axis_name="s")

    @functools.partial(
        pl.kernel,
        mesh=mesh,
        compiler_params=pltpu.CompilerParams(
            use_tc_tiling_on_sc=False, needs_layout_passes=False
        ),
        out_type=jax.ShapeDtypeStruct((B, DT, NTC, 8, W), jnp.float32),
        scratch_types=[
            pltpu.VMEM((B_PER_W, W), jnp.int32),
            pltpu.VMEM((W, D), jnp.float32),
            pltpu.VMEM((W, D), jnp.float32),
            pltpu.VMEM((W, D), jnp.float32),
            pltpu.VMEM((DT, 8, W), jnp.float32),
            pltpu.VMEM((DT, 8, W), jnp.float32),
            pltpu.SemaphoreType.DMA,
            pltpu.SemaphoreType.DMA,
            pltpu.SemaphoreType.DMA,
            pltpu.SemaphoreType.DMA,
        ],
    )
    def k(
        x_hbm, tok_hbm, pos_hbm, out_hbm,
        idx_v, rows0, rows1, pos_v, out0, out1,
        sg0, sg1, st0, st1,
    ):
        wid = lax.axis_index("s") * NC + lax.axis_index("c")
        b0 = wid * B_PER_W
        iota = lax.iota(jnp.int32, LANES)
        dt_vecs = [(c + iota) >> 3 for c in range(0, D, LANES)]
        dr_vecs = [(c + iota) & 7 for c in range(0, D, LANES)]
        rows = (rows0, rows1)
        outs = (out0, out1)
        sgs = (sg0, sg1)
        sts = (st0, st1)

        def compute(p):
            rv, ov = rows[p], outs[p]

            @plsc.parallel_loop(0, W, unroll=TUNROLL)
            def _(t):
                tvec = jnp.full((LANES,), 0, jnp.int32) + t
                for g, c in enumerate(range(0, D, LANES)):
                    v = (
                        rv[t, pl.ds(c, LANES)]
                        + pos_v[t, pl.ds(c, LANES)]
                    )
                    plsc.store_scatter(
                        ov, [dt_vecs[g], dr_vecs[g], tvec], v
                    )

        @pl.loop(0, NTC)
        def _(tc):
            pltpu.sync_copy(pos_hbm.at[pl.ds(tc * W, W)], pos_v)
            pltpu.sync_copy(
                x_hbm.at[pl.ds(b0, B_PER_W), pl.ds(tc * W, W)], idx_v
            )
            pltpu.make_async_copy(
                tok_hbm.at[idx_v.at[0]], rows[0], sgs[0]
            ).start()

            @pl.loop(0, B_PER_W // 2)
            def _(i2):
                for p in range(2):
                    i = 2 * i2 + p
                    q = 1 - p
                    pltpu.make_async_copy(
                        tok_hbm.at[idx_v.at[i]], rows[p], sgs[p]
                    ).wait()

                    @pl.when(i < B_PER_W - 1)
                    def _():
                        pltpu.make_async_copy(
                            tok_hbm.at[idx_v.at[i + 1]], rows[q], sgs[q]
                        ).start()

                    @pl.when(i >= 2)
                    def _():
                        pltpu.make_async_copy(
                            outs[p], out_hbm.at[b0 + i - 2, :, tc], sts[p]
                        ).wait()

                    compute(p)
                    pltpu.make_async_copy(
                        outs[p], out_hbm.at[b0 + i, :, tc], sts[p]
                    ).start()

            pltpu.make_async_copy(
                outs[0], out_hbm.at[b0 + B_PER_W - 2, :, tc], sts[0]
            ).wait()
            pltpu.make_async_copy(
                outs[1], out_hbm.at[b0 + B_PER_W - 1, :, tc], sts[1]
            ).wait()

    return k(x, token_table, pos_table)


def kernel(x, token_table, pos_table):
    B, T = x.shape
    D = token_table.shape[1]
    out5 = _embed(x, token_table, pos_table, B, T, D)
    out = (
        out5.transpose(0, 1, 3, 2, 4)
        .reshape(B, D, T)
        .transpose(0, 2, 1)
    )
    return out
